# baseline (speedup 1.0000x reference)
import functools

import jax
import jax.numpy as jnp
from jax import lax
from jax.experimental import pallas as pl
from jax.experimental.pallas import tpu as pltpu
from jax.experimental.pallas import tpu_sc as plsc

NC = 2
NS = 16
NW = NC * NS
C = 16
LANES = 16


def _emb_body(ids_ref, wte_ref, wpe_ref, out_ref, idx_v, rows_v, pos_v,
              gsem, psem):
    d = wte_ref.shape[1]
    nchunk = ids_ref.shape[1]
    tok_per_w = nchunk * C
    s_len = wpe_ref.shape[0]
    wid = lax.axis_index("s") * NC + lax.axis_index("c")
    base = wid * tok_per_w
    s0 = lax.rem(base, s_len)

    pltpu.sync_copy(ids_ref.at[wid], idx_v)

    slices_per_row = d // LANES

    def chunk(i, _):
        g = pltpu.async_copy(wte_ref.at[idx_v.at[i]], rows_v, gsem)
        p = pltpu.async_copy(wpe_ref.at[pl.ds(s0 + i * C, C)], pos_v, psem)
        g.wait()
        p.wait()

        def add(k, _):
            r = k // slices_per_row
            off = lax.rem(k, slices_per_row) * LANES
            rows_v[r, pl.ds(off, LANES)] = (
                rows_v[r, pl.ds(off, LANES)] + pos_v[r, pl.ds(off, LANES)])
            return 0

        lax.fori_loop(0, C * slices_per_row, add, 0)
        pltpu.sync_copy(rows_v, out_ref.at[pl.ds(base + i * C, C)])
        return 0

    lax.fori_loop(0, nchunk, chunk, 0)


def kernel(input_ids, attention_mask, wte, wpe):
    b, s = input_ids.shape
    d = wte.shape[1]
    n = b * s
    nchunk = n // (NW * C)
    ids = input_ids.reshape(NW, nchunk, C).astype(jnp.int32)

    mesh = plsc.VectorSubcoreMesh(core_axis_name="c", subcore_axis_name="s")
    run = functools.partial(
        pl.kernel,
        mesh=mesh,
        out_type=jax.ShapeDtypeStruct((n, d), jnp.float32),
        scratch_types=[
            pltpu.VMEM((nchunk, C), jnp.int32),
            pltpu.VMEM((C, d), jnp.float32),
            pltpu.VMEM((C, d), jnp.float32),
            pltpu.SemaphoreType.DMA,
            pltpu.SemaphoreType.DMA,
        ],
    )(_emb_body)
    out = run(ids, wte, wpe)
    return (attention_mask, out.reshape(b, s, d))


# speedup vs baseline: 2.5610x; 2.5610x over previous
import functools

import jax
import jax.numpy as jnp
from jax import lax
from jax.experimental import pallas as pl
from jax.experimental.pallas import tpu as pltpu
from jax.experimental.pallas import tpu_sc as plsc

NC = 2
NS = 16
NW = NC * NS
CS = 4
LANES = 16
NBUF = 3


def _emb_body(ids_ref, wte_ref, wpe_ref, out_ref, idx_v,
              rows0, rows1, rows2, pos0, pos1, pos2,
              gs0, gs1, gs2, ps0, ps1, ps2, ws0, ws1, ws2):
    rows = (rows0, rows1, rows2)
    pos = (pos0, pos1, pos2)
    gsem = (gs0, gs1, gs2)
    psem = (ps0, ps1, ps2)
    wsem = (ws0, ws1, ws2)

    d = wte_ref.shape[1]
    nchunk = ids_ref.shape[1]
    g = ids_ref.shape[2]
    nb = g // CS
    sp = nchunk * CS
    s_len = wpe_ref.shape[0]
    slices = d // LANES

    wid = lax.axis_index("s") * NC + lax.axis_index("c")
    s0 = wid * sp

    pltpu.sync_copy(ids_ref.at[wid], idx_v)

    def start_in(i):
        k = i % NBUF
        gh = pltpu.async_copy(wte_ref.at[idx_v.at[i]], rows[k], gsem[k])
        ph = pltpu.async_copy(wpe_ref.at[pl.ds(s0 + i * CS, CS)], pos[k],
                              psem[k])
        return gh, ph

    inflight = [start_in(i) for i in range(NBUF)]
    writes = [None] * NBUF

    for i in range(nchunk):
        k = i % NBUF
        gh, ph = inflight[k]
        gh.wait()
        ph.wait()

        def add_slices(m, _, k=k):
            off = m * LANES
            for j in range(CS):
                p = pos[k][j, pl.ds(off, LANES)]
                for b in range(nb):
                    plsc.addupdate(rows[k].at[b * CS + j, pl.ds(off, LANES)],
                                   p)
            return 0

        lax.fori_loop(0, slices, add_slices, 0)

        writes[k] = [
            pltpu.async_copy(
                rows[k].at[pl.ds(b * CS, CS)],
                out_ref.at[pl.ds(b * s_len + s0 + i * CS, CS)],
                wsem[k])
            for b in range(nb)
        ]

        nxt = i + NBUF
        if nxt < nchunk:
            for wh in writes[k]:
                wh.wait()
            inflight[k] = start_in(nxt)

    for k in range(NBUF):
        if writes[k] is not None:
            for wh in writes[k]:
                wh.wait()


def kernel(input_ids, attention_mask, wte, wpe):
    b, s = input_ids.shape
    d = wte.shape[1]
    n = b * s
    sp = s // NW
    nchunk = sp // CS
    g = b * CS
    ids = (input_ids.astype(jnp.int32)
           .reshape(b, NW, nchunk, CS)
           .transpose(1, 2, 0, 3)
           .reshape(NW, nchunk, g))

    mesh = plsc.VectorSubcoreMesh(core_axis_name="c", subcore_axis_name="s")
    run = functools.partial(
        pl.kernel,
        mesh=mesh,
        out_type=jax.ShapeDtypeStruct((n, d), jnp.float32),
        scratch_types=(
            [pltpu.VMEM((nchunk, g), jnp.int32)]
            + [pltpu.VMEM((g, d), jnp.float32)] * NBUF
            + [pltpu.VMEM((CS, d), jnp.float32)] * NBUF
            + [pltpu.SemaphoreType.DMA] * (3 * NBUF)
        ),
    )(_emb_body)
    out = run(ids, wte, wpe)
    return (attention_mask, out.reshape(b, s, d))


# speedup vs baseline: 2.7423x; 1.0708x over previous
import functools

import jax
import jax.numpy as jnp
from jax import lax
from jax.experimental import pallas as pl
from jax.experimental.pallas import tpu as pltpu
from jax.experimental.pallas import tpu_sc as plsc

NC = 2
NS = 16
NW = NC * NS
CS = 4
LANES = 16
NBUF = 3


def _emb_body(ids_ref, wte_ref, wpe_ref, out_ref, idx_v,
              rows0, rows1, rows2, pos0, pos1, pos2,
              gs0, gs1, gs2, ps0, ps1, ps2, ws0, ws1, ws2):
    rows = (rows0, rows1, rows2)
    pos = (pos0, pos1, pos2)
    gsem = (gs0, gs1, gs2)
    psem = (ps0, ps1, ps2)
    wsem = (ws0, ws1, ws2)

    d = wte_ref.shape[1]
    nchunk = ids_ref.shape[1]
    g = ids_ref.shape[2]
    nb = g // CS
    sp = nchunk * CS
    s_len = wpe_ref.shape[0]
    slices = d // LANES

    wid = lax.axis_index("s") * NC + lax.axis_index("c")
    s0 = wid * sp

    pltpu.sync_copy(ids_ref.at[wid], idx_v)

    def start_in(i):
        k = i % NBUF
        gh = pltpu.async_copy(wte_ref.at[idx_v.at[i]], rows[k], gsem[k])
        ph = pltpu.async_copy(wpe_ref.at[pl.ds(s0 + i * CS, CS)], pos[k],
                              psem[k])
        return gh, ph

    inflight = [start_in(i) for i in range(NBUF)]
    writes = [None] * NBUF

    for i in range(nchunk):
        k = i % NBUF
        gh, ph = inflight[k]
        gh.wait()
        ph.wait()

        def add_slices(m, _, k=k):
            off = m * LANES
            for j in range(CS):
                p = pos[k][j, pl.ds(off, LANES)]
                for b in range(nb):
                    plsc.addupdate(rows[k].at[b * CS + j, pl.ds(off, LANES)],
                                   p)
            return 0

        lax.fori_loop(0, slices, add_slices, 0)

        writes[k] = [
            pltpu.async_copy(
                rows[k].at[pl.ds(b * CS, CS)],
                out_ref.at[pl.ds(b * s_len + s0 + i * CS, CS)],
                wsem[k])
            for b in range(nb)
        ]

        prev = i - 1
        nxt = prev + NBUF
        if prev >= 0 and nxt < nchunk:
            kp = prev % NBUF
            for wh in writes[kp]:
                wh.wait()
            inflight[kp] = start_in(nxt)

    for k in range(NBUF):
        if writes[k] is not None:
            for wh in writes[k]:
                wh.wait()


def kernel(input_ids, attention_mask, wte, wpe):
    b, s = input_ids.shape
    d = wte.shape[1]
    n = b * s
    sp = s // NW
    nchunk = sp // CS
    g = b * CS
    ids = (input_ids.astype(jnp.int32)
           .reshape(b, NW, nchunk, CS)
           .transpose(1, 2, 0, 3)
           .reshape(NW, nchunk, g))

    mesh = plsc.VectorSubcoreMesh(core_axis_name="c", subcore_axis_name="s")
    run = functools.partial(
        pl.kernel,
        mesh=mesh,
        out_type=jax.ShapeDtypeStruct((n, d), jnp.float32),
        scratch_types=(
            [pltpu.VMEM((nchunk, g), jnp.int32)]
            + [pltpu.VMEM((g, d), jnp.float32)] * NBUF
            + [pltpu.VMEM((CS, d), jnp.float32)] * NBUF
            + [pltpu.SemaphoreType.DMA] * (3 * NBUF)
        ),
    )(_emb_body)
    out = run(ids, wte, wpe)
    return (attention_mask, out.reshape(b, s, d))


# speedup vs baseline: 3.0063x; 1.0963x over previous
import functools

import jax
import jax.numpy as jnp
from jax import lax
from jax.experimental import pallas as pl
from jax.experimental.pallas import tpu as pltpu
from jax.experimental.pallas import tpu_sc as plsc

NC = 2
NS = 16
NW = NC * NS
CS = 4
LANES = 16
NBUF = 3
NB = 4


def _emb_body(ids_ref, wte_ref, wpe_ref, out_ref, *scr):
    raw_v = scr[0]
    rows = tuple(tuple(scr[1 + k * NB + b] for b in range(NB))
                 for k in range(NBUF))
    pos = scr[1 + NBUF * NB:1 + NBUF * NB + NBUF]
    sems = scr[1 + NBUF * NB + NBUF:]
    gsem = sems[0:NBUF]
    psem = sems[NBUF:2 * NBUF]
    wsem = sems[2 * NBUF:3 * NBUF]

    d = wte_ref.shape[1]
    sp = raw_v.shape[1]
    nchunk = sp // CS
    s_len = wpe_ref.shape[0]
    slices = d // LANES

    wid = lax.axis_index("s") * NC + lax.axis_index("c")
    s0 = wid * sp

    for b in range(NB):
        pltpu.sync_copy(ids_ref.at[b, pl.ds(s0, sp)], raw_v.at[b])

    def start_in(i):
        k = i % NBUF
        ghs = [
            pltpu.async_copy(
                wte_ref.at[raw_v.at[b, pl.ds(i * CS, CS)]],
                rows[k][b],
                gsem[k])
            for b in range(NB)
        ]
        ph = pltpu.async_copy(wpe_ref.at[pl.ds(s0 + i * CS, CS)], pos[k],
                              psem[k])
        return ghs, ph

    inflight = [start_in(i) for i in range(NBUF)]
    writes = [None] * NBUF

    for i in range(nchunk):
        k = i % NBUF
        ghs, ph = inflight[k]
        for gh in ghs:
            gh.wait()
        ph.wait()

        def add_slices(m, _, k=k):
            off = m * LANES
            for j in range(CS):
                p = pos[k][j, pl.ds(off, LANES)]
                for b in range(NB):
                    plsc.addupdate(rows[k][b].at[j, pl.ds(off, LANES)], p)
            return 0

        lax.fori_loop(0, slices, add_slices, 0)

        writes[k] = [
            pltpu.async_copy(
                rows[k][b],
                out_ref.at[pl.ds(b * s_len + s0 + i * CS, CS)],
                wsem[k])
            for b in range(NB)
        ]

        prev = i - 1
        nxt = prev + NBUF
        if prev >= 0 and nxt < nchunk:
            kp = prev % NBUF
            for wh in writes[kp]:
                wh.wait()
            inflight[kp] = start_in(nxt)

    for k in range(NBUF):
        if writes[k] is not None:
            for wh in writes[k]:
                wh.wait()


def kernel(input_ids, attention_mask, wte, wpe):
    b, s = input_ids.shape
    d = wte.shape[1]
    n = b * s
    sp = s // NW
    ids = input_ids.astype(jnp.int32)

    mesh = plsc.VectorSubcoreMesh(core_axis_name="c", subcore_axis_name="s")
    run = functools.partial(
        pl.kernel,
        mesh=mesh,
        out_type=jax.ShapeDtypeStruct((n, d), jnp.float32),
        scratch_types=(
            [pltpu.VMEM((b, sp), jnp.int32)]
            + [pltpu.VMEM((CS, d), jnp.float32)] * (NBUF * NB)
            + [pltpu.VMEM((CS, d), jnp.float32)] * NBUF
            + [pltpu.SemaphoreType.DMA] * (3 * NBUF)
        ),
    )(_emb_body)
    out = run(ids, wte, wpe)
    return (attention_mask, out.reshape(b, s, d))
